# baseline (speedup 1.0000x reference)
import jax
import jax.numpy as jnp
from jax import lax
from jax.experimental import pallas as pl
from jax.experimental.pallas import tpu as pltpu
from jax.experimental.pallas import tpu_sc as plsc

BATCH = 16384
EMBED_DIM = 32
NUM_USERS = 1000000
NUM_ITEMS = 100000
NUM_WORKERS = 32
RPW = BATCH // NUM_WORKERS
CHUNK = 128
NCHUNK = RPW // CHUNK

U_QTR = 128 * 31
U_BLOCK = 4 * U_QTR
U_GRID = -(-NUM_USERS // U_BLOCK)
I_QTR = 128 * 7
I_BLOCK = 4 * I_QTR
I_GRID = -(-NUM_ITEMS // I_BLOCK)

_MLP_BLOCK = 2048


def _repack_body(src_ref, out_ref):
    qtr = src_ref.shape[1] // 4
    ir = lax.broadcasted_iota(jnp.int32, (EMBED_DIM, EMBED_DIM), 0)
    ic = lax.broadcasted_iota(jnp.int32, (EMBED_DIM, EMBED_DIM), 1)
    eye = (ir == ic).astype(jnp.float32)
    pieces = [
        lax.dot_general(src_ref[:, pl.ds(k * qtr, qtr)], eye,
                        (((0,), (0,)), ((), ())),
                        preferred_element_type=jnp.float32)
        for k in range(4)
    ]
    out_ref[...] = jnp.concatenate(pieces, axis=1)


def _repack(tbl_t, qtr, grid):
    return pl.pallas_call(
        _repack_body,
        grid=(grid,),
        in_specs=[pl.BlockSpec((EMBED_DIM, 4 * qtr), lambda c: (0, c))],
        out_specs=pl.BlockSpec((qtr, 4 * EMBED_DIM), lambda c: (c, 0)),
        out_shape=jax.ShapeDtypeStruct((grid * qtr, 4 * EMBED_DIM), jnp.float32),
    )(tbl_t)


def _packed_row(idx, block, qtr):
    j = idx % block
    return (idx // block) * qtr + j % qtr, j // qtr


def _sc_gather_body(user_tbl, item_tbl, u_idx, i_idx,
                    u_out, i_out,
                    uidx_v, iidx_v, rows_v, sem):
    wid = lax.axis_index("s") * 2 + lax.axis_index("c")
    base = wid * RPW

    pltpu.sync_copy(u_idx.at[wid], uidx_v)
    pltpu.sync_copy(i_idx.at[wid], iidx_v)

    for tbl, idx_v, out in ((user_tbl, uidx_v, u_out),
                            (item_tbl, iidx_v, i_out)):
        copies = []
        for j in range(NCHUNK):
            copies.append(pltpu.async_copy(
                tbl.at[idx_v.at[j]],
                rows_v.at[pl.ds(j * CHUNK, CHUNK)], sem))
        for c in copies:
            c.wait()
        pltpu.sync_copy(rows_v, out.at[pl.ds(base, RPW)])


def _sc_gather(user_tbl, item_tbl, u_idx, i_idx):
    mesh = plsc.VectorSubcoreMesh(core_axis_name="c", subcore_axis_name="s")
    fn = pl.kernel(
        _sc_gather_body,
        out_type=[
            jax.ShapeDtypeStruct((BATCH, 4 * EMBED_DIM), jnp.float32),
            jax.ShapeDtypeStruct((BATCH, 4 * EMBED_DIM), jnp.float32),
        ],
        mesh=mesh,
        scratch_types=[
            pltpu.VMEM((NCHUNK, CHUNK), jnp.int32),
            pltpu.VMEM((NCHUNK, CHUNK), jnp.int32),
            pltpu.VMEM((RPW, 4 * EMBED_DIM), jnp.float32),
            pltpu.SemaphoreType.DMA,
        ],
    )
    return fn(user_tbl, item_tbl, u_idx, i_idx)


def _select_quarter(x128, q):
    out = jnp.zeros((x128.shape[0], EMBED_DIM), jnp.float32)
    qc = q[:, None]
    for k in range(4):
        out = jnp.where(qc == k, x128[:, k * EMBED_DIM:(k + 1) * EMBED_DIM], out)
    return out


def _mlp_body(u_ref, v_ref, qu_ref, qi_ref, w1u_ref, w1i_ref, b1_ref,
              w2_ref, b2_ref, w3t_ref, b3_ref, out_ref):
    u = _select_quarter(u_ref[...], qu_ref[...])
    v = _select_quarter(v_ref[...], qi_ref[...])
    x1 = (jnp.dot(u, w1u_ref[...], preferred_element_type=jnp.float32)
          + jnp.dot(v, w1i_ref[...], preferred_element_type=jnp.float32)
          + b1_ref[...])
    h1 = jnp.maximum(x1, 0.0)
    h2 = jnp.maximum(
        jnp.dot(h1, w2_ref[...], preferred_element_type=jnp.float32)
        + b2_ref[...], 0.0)
    out_ref[...] = jnp.sum(h2 * w3t_ref[...], axis=1) + b3_ref[0]


def _mlp(u, v, qu, qi, W1u, W1i, b1, W2, b2, W3t, b3):
    grid = (BATCH // _MLP_BLOCK,)
    full = lambda i: (0, 0)
    return pl.pallas_call(
        _mlp_body,
        grid=grid,
        in_specs=[
            pl.BlockSpec((_MLP_BLOCK, 4 * EMBED_DIM), lambda i: (i, 0)),
            pl.BlockSpec((_MLP_BLOCK, 4 * EMBED_DIM), lambda i: (i, 0)),
            pl.BlockSpec((_MLP_BLOCK,), lambda i: (i,)),
            pl.BlockSpec((_MLP_BLOCK,), lambda i: (i,)),
            pl.BlockSpec((EMBED_DIM, 64), full),
            pl.BlockSpec((EMBED_DIM, 64), full),
            pl.BlockSpec((1, 64), full),
            pl.BlockSpec((64, 32), full),
            pl.BlockSpec((1, 32), full),
            pl.BlockSpec((1, 32), full),
            pl.BlockSpec((1,), lambda i: (0,)),
        ],
        out_specs=pl.BlockSpec((_MLP_BLOCK,), lambda i: (i,)),
        out_shape=jax.ShapeDtypeStruct((BATCH,), jnp.float32),
    )(u, v, qu, qi, W1u, W1i, b1, W2, b2, W3t, b3)


@jax.jit
def _run(user_indices, item_indices, user_table, item_table,
         W1, b1, W2, b2, W3, b3):
    ui = user_indices.astype(jnp.int32)
    ii = item_indices.astype(jnp.int32)
    ur, uq = _packed_row(ui, U_BLOCK, U_QTR)
    ir, iq = _packed_row(ii, I_BLOCK, I_QTR)
    u_pack = _repack(user_table.T, U_QTR, U_GRID)
    i_pack = _repack(item_table.T, I_QTR, I_GRID)
    u128, v128 = _sc_gather(u_pack, i_pack,
                            ur.reshape(NUM_WORKERS, NCHUNK, CHUNK),
                            ir.reshape(NUM_WORKERS, NCHUNK, CHUNK))
    W1u = W1[:EMBED_DIM, :]
    W1i = W1[EMBED_DIM:, :]
    pred = _mlp(u128, v128, uq, iq, W1u, W1i, b1.reshape(1, 64), W2,
                b2.reshape(1, 32), W3.reshape(1, 32), b3)
    return pred.reshape(BATCH, 1)


def kernel(user_indices, item_indices, user_table, item_table,
           W1, b1, W2, b2, W3, b3):
    return _run(user_indices, item_indices, user_table, item_table,
                W1, b1, W2, b2, W3, b3)


# speedup vs baseline: 1.0026x; 1.0026x over previous
import jax
import jax.numpy as jnp
from jax import lax
from jax.experimental import pallas as pl
from jax.experimental.pallas import tpu as pltpu
from jax.experimental.pallas import tpu_sc as plsc

BATCH = 16384
EMBED_DIM = 32
NUM_USERS = 1000000
NUM_ITEMS = 100000
NUM_WORKERS = 32
RPW = BATCH // NUM_WORKERS
CHUNK = 128
NCHUNK = RPW // CHUNK

U_QTR = 128 * 31
U_BLOCK = 4 * U_QTR
U_GRID = -(-NUM_USERS // U_BLOCK)
I_QTR = 128 * 7
I_BLOCK = 4 * I_QTR
I_GRID = -(-NUM_ITEMS // I_BLOCK)

_MLP_BLOCK = 2048


def _repack_body(src_ref, out_ref):
    qtr = src_ref.shape[1] // 4
    pieces = [src_ref[:, pl.ds(k * qtr, qtr)].T for k in range(4)]
    out_ref[...] = jnp.concatenate(pieces, axis=1)


def _repack(tbl_t, qtr, grid):
    return pl.pallas_call(
        _repack_body,
        grid=(grid,),
        in_specs=[pl.BlockSpec((EMBED_DIM, 4 * qtr), lambda c: (0, c))],
        out_specs=pl.BlockSpec((qtr, 4 * EMBED_DIM), lambda c: (c, 0)),
        out_shape=jax.ShapeDtypeStruct((grid * qtr, 4 * EMBED_DIM), jnp.float32),
    )(tbl_t)


def _packed_row(idx, block, qtr):
    j = idx % block
    return (idx // block) * qtr + j % qtr, j // qtr


def _sc_gather_body(user_tbl, item_tbl, u_idx, i_idx,
                    u_out, i_out,
                    uidx_v, iidx_v, rows_v, sem):
    wid = lax.axis_index("s") * 2 + lax.axis_index("c")
    base = wid * RPW

    pltpu.sync_copy(u_idx.at[wid], uidx_v)
    pltpu.sync_copy(i_idx.at[wid], iidx_v)

    for tbl, idx_v, out in ((user_tbl, uidx_v, u_out),
                            (item_tbl, iidx_v, i_out)):
        copies = []
        for j in range(NCHUNK):
            copies.append(pltpu.async_copy(
                tbl.at[idx_v.at[j]],
                rows_v.at[pl.ds(j * CHUNK, CHUNK)], sem))
        for c in copies:
            c.wait()
        pltpu.sync_copy(rows_v, out.at[pl.ds(base, RPW)])


def _sc_gather(user_tbl, item_tbl, u_idx, i_idx):
    mesh = plsc.VectorSubcoreMesh(core_axis_name="c", subcore_axis_name="s")
    fn = pl.kernel(
        _sc_gather_body,
        out_type=[
            jax.ShapeDtypeStruct((BATCH, 4 * EMBED_DIM), jnp.float32),
            jax.ShapeDtypeStruct((BATCH, 4 * EMBED_DIM), jnp.float32),
        ],
        mesh=mesh,
        scratch_types=[
            pltpu.VMEM((NCHUNK, CHUNK), jnp.int32),
            pltpu.VMEM((NCHUNK, CHUNK), jnp.int32),
            pltpu.VMEM((RPW, 4 * EMBED_DIM), jnp.float32),
            pltpu.SemaphoreType.DMA,
        ],
    )
    return fn(user_tbl, item_tbl, u_idx, i_idx)


def _select_quarter(x128, q):
    out = jnp.zeros((x128.shape[0], EMBED_DIM), jnp.float32)
    qc = q[:, None]
    for k in range(4):
        out = jnp.where(qc == k, x128[:, k * EMBED_DIM:(k + 1) * EMBED_DIM], out)
    return out


def _mlp_body(u_ref, v_ref, qu_ref, qi_ref, w1u_ref, w1i_ref, b1_ref,
              w2_ref, b2_ref, w3t_ref, b3_ref, out_ref):
    u = _select_quarter(u_ref[...], qu_ref[...])
    v = _select_quarter(v_ref[...], qi_ref[...])
    x1 = (jnp.dot(u, w1u_ref[...], preferred_element_type=jnp.float32)
          + jnp.dot(v, w1i_ref[...], preferred_element_type=jnp.float32)
          + b1_ref[...])
    h1 = jnp.maximum(x1, 0.0)
    h2 = jnp.maximum(
        jnp.dot(h1, w2_ref[...], preferred_element_type=jnp.float32)
        + b2_ref[...], 0.0)
    out_ref[...] = jnp.sum(h2 * w3t_ref[...], axis=1) + b3_ref[0]


def _mlp(u, v, qu, qi, W1u, W1i, b1, W2, b2, W3t, b3):
    grid = (BATCH // _MLP_BLOCK,)
    full = lambda i: (0, 0)
    return pl.pallas_call(
        _mlp_body,
        grid=grid,
        in_specs=[
            pl.BlockSpec((_MLP_BLOCK, 4 * EMBED_DIM), lambda i: (i, 0)),
            pl.BlockSpec((_MLP_BLOCK, 4 * EMBED_DIM), lambda i: (i, 0)),
            pl.BlockSpec((_MLP_BLOCK,), lambda i: (i,)),
            pl.BlockSpec((_MLP_BLOCK,), lambda i: (i,)),
            pl.BlockSpec((EMBED_DIM, 64), full),
            pl.BlockSpec((EMBED_DIM, 64), full),
            pl.BlockSpec((1, 64), full),
            pl.BlockSpec((64, 32), full),
            pl.BlockSpec((1, 32), full),
            pl.BlockSpec((1, 32), full),
            pl.BlockSpec((1,), lambda i: (0,)),
        ],
        out_specs=pl.BlockSpec((_MLP_BLOCK,), lambda i: (i,)),
        out_shape=jax.ShapeDtypeStruct((BATCH,), jnp.float32),
    )(u, v, qu, qi, W1u, W1i, b1, W2, b2, W3t, b3)


@jax.jit
def _run(user_indices, item_indices, user_table, item_table,
         W1, b1, W2, b2, W3, b3):
    ui = user_indices.astype(jnp.int32)
    ii = item_indices.astype(jnp.int32)
    ur, uq = _packed_row(ui, U_BLOCK, U_QTR)
    ir, iq = _packed_row(ii, I_BLOCK, I_QTR)
    u_pack = _repack(user_table.T, U_QTR, U_GRID)
    i_pack = _repack(item_table.T, I_QTR, I_GRID)
    u128, v128 = _sc_gather(u_pack, i_pack,
                            ur.reshape(NUM_WORKERS, NCHUNK, CHUNK),
                            ir.reshape(NUM_WORKERS, NCHUNK, CHUNK))
    W1u = W1[:EMBED_DIM, :]
    W1i = W1[EMBED_DIM:, :]
    pred = _mlp(u128, v128, uq, iq, W1u, W1i, b1.reshape(1, 64), W2,
                b2.reshape(1, 32), W3.reshape(1, 32), b3)
    return pred.reshape(BATCH, 1)


def kernel(user_indices, item_indices, user_table, item_table,
           W1, b1, W2, b2, W3, b3):
    return _run(user_indices, item_indices, user_table, item_table,
                W1, b1, W2, b2, W3, b3)
